# baseline (speedup 1.0000x reference)
import functools

import jax
import jax.numpy as jnp
import numpy as np
from jax import lax
from jax.experimental import pallas as pl
from jax.experimental.pallas import tpu as pltpu
from jax.experimental.pallas import tpu_sc as plsc

B, L, D = 4096, 200, 64
NC, NS = 2, 16
NW = NC * NS
BPW = B // NW
LANES = 16
PW = D // 2

Q = 8192
QBITS = Q.bit_length() - 1
RBLK = 4 * Q


def _pack_rows(t):
    ti = lax.bitcast_convert_type(t, jnp.int32)
    r = ti + jnp.int32(0x7FFF) + ((ti >> 16) & 1)
    lo = lax.shift_right_logical(r[:, :PW], 16)
    hi = r[:, PW:] & jnp.int32(-65536)
    return lax.bitcast_convert_type(hi | lo, jnp.float32)


def _retile_body(a_ref, b_ref, c_ref, d_ref, o_ref):
    o_ref[...] = jnp.concatenate(
        [_pack_rows(a_ref[...].T), _pack_rows(b_ref[...].T),
         _pack_rows(c_ref[...].T), _pack_rows(d_ref[...].T)], axis=1)


@functools.partial(jax.jit, static_argnums=(1,))
def _tc_retile(table_t, n_rows):
    nb = (n_rows + RBLK - 1) // RBLK
    max_ib = (n_rows - 1) // Q
    return pl.pallas_call(
        _retile_body,
        grid=(nb,),
        in_specs=[
            pl.BlockSpec((D, Q), lambda i, q=q: (0, jnp.minimum(4 * i + q, max_ib)))
            for q in range(4)
        ],
        out_specs=pl.BlockSpec((Q, 2 * D), lambda i: (i, 0)),
        out_shape=jax.ShapeDtypeStruct((nb * Q, 2 * D), jnp.float32),
    )(table_t, table_t, table_t, table_t)


def _remap_idx(g):
    return (4 * (((g >> (QBITS + 2)) << QBITS) | (g & (Q - 1)))
            | ((g >> QBITS) & 3))


def _sc_gather_sum_body(item_hbm, user_hbm, idx_hbm, uids_hbm, out_hbm,
                        idx_v, uids_v, self_v, rows0_v, rows1_v, out_v,
                        sem0, sem1, usem):
    wid = lax.axis_index("s") * NC + lax.axis_index("c")
    base = wid * BPW

    pltpu.sync_copy(idx_hbm.at[pl.ds(base, BPW)], idx_v)
    pltpu.sync_copy(uids_hbm.at[pl.ds(base, BPW)], uids_v)
    user_cp = pltpu.async_copy(user_hbm.at[uids_v], self_v, usem)

    def start(b, rows_v, sem):
        pltpu.async_copy(item_hbm.at[idx_v.at[b]], rows_v, sem)

    def halves(packed):
        v = plsc.bitcast(packed, jnp.int32)
        lo = plsc.bitcast(v << 16, jnp.float32)
        hi = plsc.bitcast(v & jnp.int32(-65536), jnp.float32)
        return lo, hi

    def accum_row(b, rows_v):
        def accum(j, accs):
            out = []
            for h in range(PW // LANES):
                lo, hi = halves(rows_v[j, pl.ds(h * LANES, LANES)])
                out += [accs[2 * h] + lo, accs[2 * h + 1] + hi]
            return tuple(out)

        init = []
        for h in range(PW // LANES):
            lo, hi = halves(self_v[b, pl.ds(h * LANES, LANES)])
            init += [lo, hi]
        accs = lax.fori_loop(0, L, accum, tuple(init), unroll=8)
        for a in range(4):
            out_v[b, pl.ds(a * LANES, LANES)] = accs[a]

    start(0, rows0_v, sem0)
    user_cp.wait()

    def outer(k, carry):
        start(2 * k + 1, rows1_v, sem1)
        pltpu.make_async_copy(
            item_hbm.at[idx_v.at[0]], rows0_v, sem0).wait()
        accum_row(2 * k, rows0_v)

        @pl.when(k < BPW // 2 - 1)
        def _():
            start(2 * k + 2, rows0_v, sem0)

        pltpu.make_async_copy(
            item_hbm.at[idx_v.at[0]], rows1_v, sem1).wait()
        accum_row(2 * k + 1, rows1_v)
        return carry

    lax.fori_loop(0, BPW // 2, outer, 0)
    pltpu.sync_copy(out_v, out_hbm.at[pl.ds(base, BPW)])


@jax.jit
def _sc_gather_sum(u_item_pad, uids, item_table, user_table):
    mesh = plsc.VectorSubcoreMesh(core_axis_name="c", subcore_axis_name="s")
    return pl.kernel(
        _sc_gather_sum_body,
        out_type=jax.ShapeDtypeStruct((B, D), jnp.float32),
        mesh=mesh,
        scratch_types=[
            pltpu.VMEM((BPW, L), jnp.int32),
            pltpu.VMEM((BPW,), jnp.int32),
            pltpu.VMEM((BPW, PW), jnp.float32),
            pltpu.VMEM((L, PW), jnp.float32),
            pltpu.VMEM((L, PW), jnp.float32),
            pltpu.VMEM((BPW, D), jnp.float32),
            pltpu.SemaphoreType.DMA,
            pltpu.SemaphoreType.DMA,
            pltpu.SemaphoreType.DMA,
        ],
        compiler_params=pltpu.CompilerParams(use_tc_tiling_on_sc=False,
                                             needs_layout_passes=False),
    )(item_table, user_table, u_item_pad, uids)


_PERM = np.concatenate([
    np.arange(0, 16), np.arange(32, 48),
    np.arange(16, 32), np.arange(48, 64),
])


def _mm_body(s_ref, w_ref, b_ref, o_ref):
    x = s_ref[...]
    y = lax.dot_general(x, w_ref[...], (((1,), (1,)), ((), ())),
                        preferred_element_type=jnp.float32)
    y = y + b_ref[...]
    o_ref[...] = jnp.where(y > 0, y, jnp.exp(jnp.minimum(y, 0.0)) - 1.0)


@jax.jit
def _mm_elu(s, W, b2d):
    blk = 512
    return pl.pallas_call(
        _mm_body,
        grid=(B // blk,),
        in_specs=[
            pl.BlockSpec((blk, D), lambda i: (i, 0)),
            pl.BlockSpec((D, D), lambda i: (0, 0)),
            pl.BlockSpec((1, D), lambda i: (0, 0)),
        ],
        out_specs=pl.BlockSpec((blk, D), lambda i: (i, 0)),
        out_shape=jax.ShapeDtypeStruct((B, D), jnp.float32),
    )(s, W, b2d)


def kernel(uids, u_item_pad, item_table, user_table, W, b):
    n_items, n_users = item_table.shape[0], user_table.shape[0]
    item_r = _tc_retile(item_table.T, n_items)
    user_r = _tc_retile(user_table.T, n_users)
    item_pk = item_r.reshape(item_r.shape[0] * 4, PW)
    user_pk = user_r.reshape(user_r.shape[0] * 4, PW)
    idx = _remap_idx(u_item_pad.astype(jnp.int32))
    uid = _remap_idx(uids.astype(jnp.int32))
    s = _sc_gather_sum(idx, uid, item_pk, user_pk)
    return _mm_elu(s, W[:, _PERM], b.reshape(1, D))


# speedup vs baseline: 1.3644x; 1.3644x over previous
import functools

import jax
import jax.numpy as jnp
import numpy as np
from jax import lax
from jax.experimental import pallas as pl
from jax.experimental.pallas import tpu as pltpu
from jax.experimental.pallas import tpu_sc as plsc

B, L, D = 4096, 200, 64
NC, NS = 2, 16
NW = NC * NS
BPW = B // NW
LANES = 16
PW = D // 2

Q = 8192
QBITS = Q.bit_length() - 1
RBLK = 4 * Q


def _pack_rows(t):
    ti = lax.bitcast_convert_type(t, jnp.int32) + jnp.int32(0x8000)
    w = lax.shift_right_logical(ti[:PW, :], 16) | (ti[PW:, :] & jnp.int32(-65536))
    return lax.bitcast_convert_type(w, jnp.float32).T


def _retile_body(a_ref, b_ref, c_ref, d_ref, o_ref):
    o_ref[...] = jnp.concatenate(
        [_pack_rows(a_ref[...]), _pack_rows(b_ref[...]),
         _pack_rows(c_ref[...]), _pack_rows(d_ref[...])], axis=1)


@functools.partial(jax.jit, static_argnums=(1,))
def _tc_retile(table_t, n_rows):
    nb = (n_rows + RBLK - 1) // RBLK
    max_ib = (n_rows - 1) // Q
    return pl.pallas_call(
        _retile_body,
        grid=(nb,),
        in_specs=[
            pl.BlockSpec((D, Q), lambda i, q=q: (0, jnp.minimum(4 * i + q, max_ib)))
            for q in range(4)
        ],
        out_specs=pl.BlockSpec((Q, 2 * D), lambda i: (i, 0)),
        out_shape=jax.ShapeDtypeStruct((nb * Q, 2 * D), jnp.float32),
    )(table_t, table_t, table_t, table_t)


def _remap_idx(g):
    return (4 * (((g >> (QBITS + 2)) << QBITS) | (g & (Q - 1)))
            | ((g >> QBITS) & 3))


def _sc_gather_sum_body(item_hbm, user_hbm, idx_hbm, uids_hbm, out_hbm,
                        idx_v, uids_v, self_v, rows0_v, rows1_v, out_v,
                        sem0, sem1, usem):
    wid = lax.axis_index("s") * NC + lax.axis_index("c")
    base = wid * BPW

    pltpu.sync_copy(idx_hbm.at[pl.ds(base, BPW)], idx_v)
    pltpu.sync_copy(uids_hbm.at[pl.ds(base, BPW)], uids_v)
    user_cp = pltpu.async_copy(user_hbm.at[uids_v], self_v, usem)

    def start(b, rows_v, sem):
        pltpu.async_copy(item_hbm.at[idx_v.at[b]], rows_v, sem)

    def halves(packed):
        v = plsc.bitcast(packed, jnp.int32)
        lo = plsc.bitcast(v << 16, jnp.float32)
        hi = plsc.bitcast(v & jnp.int32(-65536), jnp.float32)
        return lo, hi

    def accum_row(b, rows_v):
        def accum(j, accs):
            out = []
            for h in range(PW // LANES):
                lo, hi = halves(rows_v[j, pl.ds(h * LANES, LANES)])
                out += [accs[2 * h] + lo, accs[2 * h + 1] + hi]
            return tuple(out)

        init = []
        for h in range(PW // LANES):
            lo, hi = halves(self_v[b, pl.ds(h * LANES, LANES)])
            init += [lo, hi]
        accs = lax.fori_loop(0, L, accum, tuple(init), unroll=8)
        for a in range(4):
            out_v[b, pl.ds(a * LANES, LANES)] = accs[a]

    start(0, rows0_v, sem0)
    user_cp.wait()

    def outer(k, carry):
        start(2 * k + 1, rows1_v, sem1)
        pltpu.make_async_copy(
            item_hbm.at[idx_v.at[0]], rows0_v, sem0).wait()
        accum_row(2 * k, rows0_v)

        @pl.when(k < BPW // 2 - 1)
        def _():
            start(2 * k + 2, rows0_v, sem0)

        pltpu.make_async_copy(
            item_hbm.at[idx_v.at[0]], rows1_v, sem1).wait()
        accum_row(2 * k + 1, rows1_v)
        return carry

    lax.fori_loop(0, BPW // 2, outer, 0)
    pltpu.sync_copy(out_v, out_hbm.at[pl.ds(base, BPW)])


@jax.jit
def _sc_gather_sum(u_item_pad, uids, item_table, user_table):
    mesh = plsc.VectorSubcoreMesh(core_axis_name="c", subcore_axis_name="s")
    return pl.kernel(
        _sc_gather_sum_body,
        out_type=jax.ShapeDtypeStruct((B, D), jnp.float32),
        mesh=mesh,
        scratch_types=[
            pltpu.VMEM((BPW, L), jnp.int32),
            pltpu.VMEM((BPW,), jnp.int32),
            pltpu.VMEM((BPW, PW), jnp.float32),
            pltpu.VMEM((L, PW), jnp.float32),
            pltpu.VMEM((L, PW), jnp.float32),
            pltpu.VMEM((BPW, D), jnp.float32),
            pltpu.SemaphoreType.DMA,
            pltpu.SemaphoreType.DMA,
            pltpu.SemaphoreType.DMA,
        ],
        compiler_params=pltpu.CompilerParams(use_tc_tiling_on_sc=False,
                                             needs_layout_passes=False),
    )(item_table, user_table, u_item_pad, uids)


_PERM = np.concatenate([
    np.arange(0, 16), np.arange(32, 48),
    np.arange(16, 32), np.arange(48, 64),
])


def _mm_body(s_ref, w_ref, b_ref, o_ref):
    x = s_ref[...]
    y = lax.dot_general(x, w_ref[...], (((1,), (1,)), ((), ())),
                        preferred_element_type=jnp.float32)
    y = y + b_ref[...]
    o_ref[...] = jnp.where(y > 0, y, jnp.exp(jnp.minimum(y, 0.0)) - 1.0)


@jax.jit
def _mm_elu(s, W, b2d):
    blk = 512
    return pl.pallas_call(
        _mm_body,
        grid=(B // blk,),
        in_specs=[
            pl.BlockSpec((blk, D), lambda i: (i, 0)),
            pl.BlockSpec((D, D), lambda i: (0, 0)),
            pl.BlockSpec((1, D), lambda i: (0, 0)),
        ],
        out_specs=pl.BlockSpec((blk, D), lambda i: (i, 0)),
        out_shape=jax.ShapeDtypeStruct((B, D), jnp.float32),
    )(s, W, b2d)


def kernel(uids, u_item_pad, item_table, user_table, W, b):
    n_items, n_users = item_table.shape[0], user_table.shape[0]
    item_r = _tc_retile(item_table.T, n_items)
    user_r = _tc_retile(user_table.T, n_users)
    item_pk = item_r.reshape(item_r.shape[0] * 4, PW)
    user_pk = user_r.reshape(user_r.shape[0] * 4, PW)
    idx = _remap_idx(u_item_pad.astype(jnp.int32))
    uid = _remap_idx(uids.astype(jnp.int32))
    s = _sc_gather_sum(idx, uid, item_pk, user_pk)
    return _mm_elu(s, W[:, _PERM], b.reshape(1, D))
